# baseline (speedup 1.0000x reference)
import functools

import jax
import jax.numpy as jnp
from jax import lax
from jax.experimental import pallas as pl
from jax.experimental.pallas import tpu as pltpu
from jax.experimental.pallas import tpu_sc as plsc

_NB = 5


def _build(B, S, V, D):
    NC, NS = 2, 16
    NW = NC * NS
    BW = B // NW
    assert BW == 128 and D == 32 and S % _NB == 0
    scale = 1.0 / B

    mesh = plsc.VectorSubcoreMesh(core_axis_name="c", subcore_axis_name="s")

    @functools.partial(
        pl.kernel,
        mesh=mesh,
        out_type=jax.ShapeDtypeStruct((D, B), jnp.float32),
        scratch_types=[
            pltpu.VMEM((S, BW), jnp.int32),
            [pltpu.VMEM((D, BW), jnp.float32)] * _NB,
            pltpu.VMEM((D, BW), jnp.float32),
            [pltpu.SemaphoreType.DMA] * _NB,
        ],
        compiler_params=pltpu.CompilerParams(use_tc_tiling_on_sc=False),
    )
    def k(docT_hbm, tT_hbm, outT_hbm, idx_v, chunks, acc_v, sems):
        wid = lax.axis_index("s") * NC + lax.axis_index("c")
        col0 = wid * BW
        pltpu.sync_copy(docT_hbm.at[:, pl.ds(col0, BW)], idx_v)

        zero = jnp.zeros((16,), jnp.float32)
        for d in range(D):
            for q in range(BW // 16):
                acc_v[d, pl.ds(16 * q, 16)] = zero

        def fire(t, buf, sem):
            for d in range(D):
                pltpu.async_copy(
                    tT_hbm.at[pl.ds(d * V, V)].at[idx_v.at[t]], buf.at[d], sem)

        for b in range(_NB):
            fire(b, chunks[b], sems[b])

        def body(i, _):
            t0 = i * _NB
            for b in range(_NB):
                t = t0 + b
                cv = chunks[b]
                for d in range(D):
                    pltpu.make_async_copy(
                        tT_hbm.at[pl.ds(d * V, V)].at[idx_v.at[t]],
                        cv.at[d], sems[b]).wait()
                for d in range(D):
                    for q in range(BW // 16):
                        plsc.addupdate(
                            acc_v.at[d, pl.ds(16 * q, 16)],
                            cv[d, pl.ds(16 * q, 16)])

                nt = t + _NB

                @pl.when(nt < S)
                def _():
                    fire(nt, cv, sems[b])

            return 0

        lax.fori_loop(0, S // _NB, body, 0)

        for d in range(D):
            for q in range(BW // 16):
                acc_v[d, pl.ds(16 * q, 16)] = (
                    acc_v[d, pl.ds(16 * q, 16)] * scale)
        pltpu.sync_copy(acc_v, outT_hbm.at[:, pl.ds(col0, BW)])

    return k


def kernel(document, table):
    B, S = document.shape
    V, D = table.shape
    outT = _build(B, S, V, D)(document.T, table.T.reshape(-1))
    return outT.T


# speedup vs baseline: 4.2502x; 4.2502x over previous
import functools

import jax
import jax.numpy as jnp
import numpy as np
from jax import lax
from jax.experimental import pallas as pl
from jax.experimental.pallas import tpu as pltpu
from jax.experimental.pallas import tpu_sc as plsc

_NB = 4


def _build(B, S, V, D):
    NC, NS = 2, 16
    NW = NC * NS
    SP = S + (-S) % 4
    CW = 2 * SP
    assert CW <= 128 and D == 32 and B % (2 * NW * _NB) == 0
    CPW = B // (2 * NW)
    RPW = B // NW
    scale = 1.0 / B

    mesh = plsc.VectorSubcoreMesh(core_axis_name="c", subcore_axis_name="s")

    @functools.partial(
        pl.kernel,
        mesh=mesh,
        out_type=jax.ShapeDtypeStruct((B, D), jnp.float32),
        scratch_types=[
            pltpu.VMEM((CPW, CW), jnp.int32),
            [pltpu.VMEM((CW, D), jnp.bfloat16)] * _NB,
            pltpu.VMEM((RPW, D), jnp.float32),
            [pltpu.SemaphoreType.DMA] * _NB,
        ],
        compiler_params=pltpu.CompilerParams(
            use_tc_tiling_on_sc=False, needs_layout_passes=False),
    )
    def k(doc_hbm, table_hbm, out_hbm, idx_v, rows, out_v, sems):
        wid = lax.axis_index("s") * NC + lax.axis_index("c")
        pltpu.sync_copy(doc_hbm.at[pl.ds(wid * CPW, CPW)], idx_v)

        for b in range(_NB):
            pltpu.async_copy(table_hbm.at[idx_v.at[b]], rows[b], sems[b])

        def body(i, _):
            j0 = i * _NB
            for b in range(_NB):
                j = j0 + b
                rv = rows[b]
                pltpu.make_async_copy(
                    table_hbm.at[idx_v.at[j]], rv, sems[b]).wait()
                for h in range(2):
                    base = h * SP
                    ev = [None, None]
                    od = [None, None]
                    for t in range(S):
                        a, bb = plsc.unpack(
                            rv[base + t, :], format=plsc.PackFormat.INTERLEAVED)
                        tgt = ev if t % 2 == 0 else od
                        tgt[0] = a if tgt[0] is None else tgt[0] + a
                        tgt[1] = bb if tgt[1] is None else tgt[1] + bb
                    out_v[2 * j + h, pl.ds(0, 16)] = (ev[0] + od[0]) * scale
                    out_v[2 * j + h, pl.ds(16, 16)] = (ev[1] + od[1]) * scale

                nj = j + _NB

                @pl.when(nj < CPW)
                def _():
                    pltpu.async_copy(table_hbm.at[idx_v.at[nj]], rv, sems[b])

            return 0

        lax.fori_loop(0, CPW // _NB, body, 0)
        pltpu.sync_copy(out_v, out_hbm.at[pl.ds(wid * RPW, RPW)])

    return k


def kernel(document, table):
    B, S = document.shape
    V, D = table.shape
    SP = S + (-S) % 4
    doc_p = jnp.pad(document, ((0, 0), (0, SP - S)))
    doc2 = doc_p.reshape(B // 2, 2 * SP)
    tb = table.astype(jnp.bfloat16)
    out_k = _build(B, S, V, D)(doc2, tb)
    perm = np.empty(D, dtype=np.int32)
    perm[0::2] = np.arange(D // 2)
    perm[1::2] = D // 2 + np.arange(D // 2)
    return out_k[:, perm]


# speedup vs baseline: 4.6403x; 1.0918x over previous
import functools

import jax
import jax.numpy as jnp
from jax import lax
from jax.experimental import pallas as pl
from jax.experimental.pallas import tpu as pltpu
from jax.experimental.pallas import tpu_sc as plsc

_NB = 4


def _build(B, S, V, D):
    NC, NS = 2, 16
    NW = NC * NS
    SP = S + (-S) % 4
    CW = 2 * SP
    assert CW <= 128 and D == 32 and B % (2 * NW * _NB) == 0
    CPW = B // (2 * NW)
    RPW = B // NW
    scale = 1.0 / B

    mesh = plsc.VectorSubcoreMesh(core_axis_name="c", subcore_axis_name="s")

    @functools.partial(
        pl.kernel,
        mesh=mesh,
        out_type=jax.ShapeDtypeStruct((B, D), jnp.float32),
        scratch_types=[
            pltpu.VMEM((CPW, CW), jnp.int32),
            [pltpu.VMEM((CW, D), jnp.float32)] * _NB,
            pltpu.VMEM((RPW, D), jnp.float32),
            [pltpu.SemaphoreType.DMA] * _NB,
        ],
        compiler_params=pltpu.CompilerParams(use_tc_tiling_on_sc=False),
    )
    def k(doc_hbm, table_hbm, out_hbm, idx_v, rows, out_v, sems):
        wid = lax.axis_index("s") * NC + lax.axis_index("c")
        pltpu.sync_copy(doc_hbm.at[pl.ds(wid * CPW, CPW)], idx_v)

        for b in range(_NB):
            pltpu.async_copy(table_hbm.at[idx_v.at[b]], rows[b], sems[b])

        def body(i, _):
            j0 = i * _NB
            for b in range(_NB):
                j = j0 + b
                rv = rows[b]
                pltpu.make_async_copy(
                    table_hbm.at[idx_v.at[j]], rv, sems[b]).wait()
                for h in range(2):
                    base = h * SP
                    ev = [None, None]
                    od = [None, None]
                    for t in range(S):
                        tgt = ev if t % 2 == 0 else od
                        for d in range(2):
                            v = rv[base + t, pl.ds(16 * d, 16)]
                            tgt[d] = v if tgt[d] is None else tgt[d] + v
                    out_v[2 * j + h, pl.ds(0, 16)] = (ev[0] + od[0]) * scale
                    out_v[2 * j + h, pl.ds(16, 16)] = (ev[1] + od[1]) * scale

                nj = j + _NB

                @pl.when(nj < CPW)
                def _():
                    pltpu.async_copy(table_hbm.at[idx_v.at[nj]], rv, sems[b])

            return 0

        lax.fori_loop(0, CPW // _NB, body, 0)
        pltpu.sync_copy(out_v, out_hbm.at[pl.ds(wid * RPW, RPW)])

    return k


def kernel(document, table):
    B, S = document.shape
    V, D = table.shape
    SP = S + (-S) % 4
    doc_p = jnp.pad(document, ((0, 0), (0, SP - S)))
    doc2 = doc_p.reshape(B // 2, 2 * SP)
    return _build(B, S, V, D)(doc2, table)
